# Initial kernel scaffold; baseline (speedup 1.0000x reference)
import jax
import jax.numpy as jnp
from jax.experimental import pallas as pl


def kernel(idx, targets, table):
    raise NotImplementedError("write your pallas kernel here")


# speedup vs baseline: 1.4126x; 1.4126x over previous
import functools

import jax
import jax.numpy as jnp
from jax import lax
from jax.experimental import pallas as pl
from jax.experimental.pallas import tpu as pltpu
from jax.experimental.pallas import tpu_sc as plsc

VOCAB = 1000
BT = 4096 * 50
NC, NS, L = 2, 16, 16
NW = NC * NS
PER_W = BT // NW
CHUNK = 32
NCHUNK = PER_W // CHUNK


def _logz_body(t_ref, o_ref):
    x = t_ref[...]
    m = jnp.max(x, axis=1)
    s = jnp.sum(jnp.exp(x - m[:, None]), axis=1)
    o_ref[...] = m + jnp.log(s)


def _row_logsumexp(table):
    return pl.pallas_call(
        _logz_body,
        out_shape=jax.ShapeDtypeStruct((VOCAB,), jnp.float32),
    )(table)


def _sc_body(table_hbm, idx_hbm, tgt_hbm, logz_hbm,
             out_hbm, part_hbm,
             idx_v, tgt_v, logz_v, rows_v, acc_v, sem):
    wid = lax.axis_index("s") * NC + lax.axis_index("c")
    base = wid * PER_W

    pltpu.sync_copy(idx_hbm.at[pl.ds(base, PER_W)], idx_v)
    pltpu.sync_copy(tgt_hbm.at[pl.ds(base, PER_W)], tgt_v)
    pltpu.sync_copy(logz_hbm, logz_v)

    def chunk_step(c, acc):
        cp = pltpu.async_copy(
            table_hbm.at[idx_v.at[pl.ds(c * CHUNK, CHUNK)]], rows_v, sem)
        cp.wait()
        for j in range(CHUNK // L):
            off = c * CHUNK + j * L
            ivec = idx_v[pl.ds(off, L)]
            tvec = tgt_v[pl.ds(off, L)]
            lz = plsc.load_gather(logz_v, [ivec])
            rowids = lax.iota(jnp.int32, L) + j * L
            pk = plsc.load_gather(rows_v, [rowids, tvec])
            acc = acc + (lz - pk)
        pltpu.sync_copy(rows_v, out_hbm.at[pl.ds(base + c * CHUNK, CHUNK)])
        return acc

    acc = lax.fori_loop(0, NCHUNK, chunk_step, jnp.zeros((L,), jnp.float32))
    acc_v[...] = acc
    pltpu.sync_copy(acc_v, part_hbm.at[wid])


def _sc_gather(table, idx_flat, tgt_flat, logz):
    mesh = plsc.VectorSubcoreMesh(core_axis_name="c", subcore_axis_name="s")
    f = pl.kernel(
        _sc_body,
        out_type=(
            jax.ShapeDtypeStruct((BT, VOCAB), jnp.float32),
            jax.ShapeDtypeStruct((NW, L), jnp.float32),
        ),
        mesh=mesh,
        compiler_params=pltpu.CompilerParams(
            needs_layout_passes=False, use_tc_tiling_on_sc=False),
        scratch_types=[
            pltpu.VMEM((PER_W,), jnp.int32),
            pltpu.VMEM((PER_W,), jnp.int32),
            pltpu.VMEM((VOCAB,), jnp.float32),
            pltpu.VMEM((CHUNK, VOCAB), jnp.float32),
            pltpu.VMEM((L,), jnp.float32),
            pltpu.SemaphoreType.DMA,
        ],
    )
    return f(table, idx_flat, tgt_flat, logz)


def kernel(idx, targets, table):
    idx_flat = idx.reshape(-1).astype(jnp.int32)
    tgt_flat = targets.reshape(-1).astype(jnp.int32)
    logz = _row_logsumexp(table)
    logits2, partials = _sc_gather(table, idx_flat, tgt_flat, logz)
    loss = jnp.sum(partials) / BT
    return (logits2, loss)


# speedup vs baseline: 1.5045x; 1.0650x over previous
import functools

import jax
import jax.numpy as jnp
from jax import lax
from jax.experimental import pallas as pl
from jax.experimental.pallas import tpu as pltpu
from jax.experimental.pallas import tpu_sc as plsc

VOCAB = 1000
BT = 4096 * 50
NC, NS, L = 2, 16, 16
NW = NC * NS
PER_W = BT // NW
CHUNK = 32
NCHUNK = PER_W // CHUNK


def _logz_body(t_ref, o_ref):
    x = t_ref[...]
    m = jnp.max(x, axis=1)
    s = jnp.sum(jnp.exp(x - m[:, None]), axis=1)
    o_ref[...] = m + jnp.log(s)


def _row_logsumexp(table):
    return pl.pallas_call(
        _logz_body,
        out_shape=jax.ShapeDtypeStruct((VOCAB,), jnp.float32),
    )(table)


NBUF = 2


def _sc_body(table_hbm, idx_hbm, tgt_hbm, logz_hbm,
             out_hbm, part_hbm,
             idx_v, tgt_v, logz_v, rows0_v, rows1_v, acc_v, sem0, sem1):
    wid = lax.axis_index("s") * NC + lax.axis_index("c")
    base = wid * PER_W
    rows_bufs = (rows0_v, rows1_v)
    sems = (sem0, sem1)

    pltpu.sync_copy(idx_hbm.at[pl.ds(base, PER_W)], idx_v)
    pltpu.sync_copy(tgt_hbm.at[pl.ds(base, PER_W)], tgt_v)
    pltpu.sync_copy(logz_hbm, logz_v)

    def gather(c, b):
        return pltpu.async_copy(
            table_hbm.at[idx_v.at[pl.ds(c * CHUNK, CHUNK)]],
            rows_bufs[b], sems[b])

    for b in range(NBUF):
        gather(b, b)

    def outer_step(g, acc):
        for b in range(NBUF):
            c = g * NBUF + b
            rows_v = rows_bufs[b]
            pltpu.make_async_copy(
                table_hbm.at[idx_v.at[pl.ds(c * CHUNK, CHUNK)]],
                rows_v, sems[b]).wait()
            for j in range(CHUNK // L):
                off = c * CHUNK + j * L
                ivec = idx_v[pl.ds(off, L)]
                tvec = tgt_v[pl.ds(off, L)]
                lz = plsc.load_gather(logz_v, [ivec])
                rowids = lax.iota(jnp.int32, L) + j * L
                pk = plsc.load_gather(rows_v, [rowids, tvec])
                acc = acc + (lz - pk)
            pltpu.sync_copy(rows_v, out_hbm.at[pl.ds(base + c * CHUNK, CHUNK)])

            @pl.when(c + NBUF < NCHUNK)
            def _():
                gather(c + NBUF, b)
        return acc

    acc = lax.fori_loop(0, NCHUNK // NBUF, outer_step,
                        jnp.zeros((L,), jnp.float32))
    acc_v[...] = acc
    pltpu.sync_copy(acc_v, part_hbm.at[wid])


def _sc_gather(table, idx_flat, tgt_flat, logz):
    mesh = plsc.VectorSubcoreMesh(core_axis_name="c", subcore_axis_name="s")
    f = pl.kernel(
        _sc_body,
        out_type=(
            jax.ShapeDtypeStruct((BT, VOCAB), jnp.float32),
            jax.ShapeDtypeStruct((NW, L), jnp.float32),
        ),
        mesh=mesh,
        compiler_params=pltpu.CompilerParams(
            needs_layout_passes=False, use_tc_tiling_on_sc=False),
        scratch_types=[
            pltpu.VMEM((PER_W,), jnp.int32),
            pltpu.VMEM((PER_W,), jnp.int32),
            pltpu.VMEM((VOCAB,), jnp.float32),
            pltpu.VMEM((CHUNK, VOCAB), jnp.float32),
            pltpu.VMEM((CHUNK, VOCAB), jnp.float32),
            pltpu.VMEM((L,), jnp.float32),
            pltpu.SemaphoreType.DMA,
            pltpu.SemaphoreType.DMA,
        ],
    )
    return f(table, idx_flat, tgt_flat, logz)


def kernel(idx, targets, table):
    idx_flat = idx.reshape(-1).astype(jnp.int32)
    tgt_flat = targets.reshape(-1).astype(jnp.int32)
    logz = _row_logsumexp(table)
    logits2, partials = _sc_gather(table, idx_flat, tgt_flat, logz)
    loss = jnp.sum(partials) / BT
    return (logits2, loss)


# speedup vs baseline: 1.5953x; 1.0604x over previous
import jax
import jax.numpy as jnp
from jax import lax
from jax.experimental import pallas as pl
from jax.experimental.pallas import tpu as pltpu
from jax.experimental.pallas import tpu_sc as plsc

VOCAB = 1000
VPAD = 1024
BT = 4096 * 50
NC, NS, L = 2, 16, 16
NW = NC * NS
PER_W = BT // NW
CHUNK = 32
NCHUNK = PER_W // CHUNK
NBUF = 2


def _logz_body(t_ref, o_ref):
    x = t_ref[...]
    m = jnp.max(x, axis=1)
    s = jnp.sum(jnp.exp(x - m[:, None]), axis=1)
    o_ref[...] = m + jnp.log(s)


def _row_logsumexp(table):
    return pl.pallas_call(
        _logz_body,
        out_shape=jax.ShapeDtypeStruct((VOCAB,), jnp.float32),
    )(table)


def _gather_body(tablep_hbm, idx_hbm, outp_hbm,
                 idx_v, rows0_v, rows1_v, sem0, sem1):
    wid = lax.axis_index("s") * NC + lax.axis_index("c")
    base = wid * PER_W
    rows_bufs = (rows0_v, rows1_v)
    sems = (sem0, sem1)

    pltpu.sync_copy(idx_hbm.at[pl.ds(base, PER_W)], idx_v)

    def gather(c, b):
        return pltpu.async_copy(
            tablep_hbm.at[idx_v.at[pl.ds(c * CHUNK, CHUNK)]],
            rows_bufs[b], sems[b])

    for b in range(NBUF):
        gather(b, b)

    def outer_step(g, carry):
        for b in range(NBUF):
            c = g * NBUF + b
            rows_v = rows_bufs[b]
            pltpu.make_async_copy(
                tablep_hbm.at[idx_v.at[pl.ds(c * CHUNK, CHUNK)]],
                rows_v, sems[b]).wait()
            pltpu.sync_copy(rows_v, outp_hbm.at[pl.ds(base + c * CHUNK, CHUNK)])

            @pl.when(c + NBUF < NCHUNK)
            def _():
                gather(c + NBUF, b)
        return carry

    lax.fori_loop(0, NCHUNK // NBUF, outer_step, 0)


def _sc_gather(tablep, idx_flat):
    mesh = plsc.VectorSubcoreMesh(core_axis_name="c", subcore_axis_name="s")
    f = pl.kernel(
        _gather_body,
        out_type=jax.ShapeDtypeStruct((BT, VPAD), jnp.float32),
        mesh=mesh,
        compiler_params=pltpu.CompilerParams(needs_layout_passes=False),
        scratch_types=[
            pltpu.VMEM((PER_W,), jnp.int32),
            pltpu.VMEM((CHUNK, VPAD), jnp.float32),
            pltpu.VMEM((CHUNK, VPAD), jnp.float32),
            pltpu.SemaphoreType.DMA,
            pltpu.SemaphoreType.DMA,
        ],
    )
    return f(tablep, idx_flat)


GRP = 128
NGRP = PER_W // GRP


def _loss_body(tflat_hbm, idx_hbm, tgt_hbm, logz_hbm, part_hbm,
               idx_v, tgt_v, fi_v, picked_v, logz_v, acc_v, sem):
    wid = lax.axis_index("s") * NC + lax.axis_index("c")
    base = wid * PER_W

    pltpu.sync_copy(idx_hbm.at[pl.ds(base, PER_W)], idx_v)
    pltpu.sync_copy(tgt_hbm.at[pl.ds(base, PER_W)], tgt_v)
    pltpu.sync_copy(logz_hbm, logz_v)

    def fi_step(g, carry):
        iv = idx_v[pl.ds(g * L, L)]
        tv = tgt_v[pl.ds(g * L, L)]
        fi_v[pl.ds(g * L, L)] = iv * VOCAB + tv
        return carry

    lax.fori_loop(0, PER_W // L, fi_step, 0)

    def pick_step(k, carry):
        pltpu.async_copy(
            tflat_hbm.at[fi_v.at[pl.ds(k * GRP, GRP)]],
            picked_v.at[pl.ds(k * GRP, GRP)], sem).wait()
        return carry

    lax.fori_loop(0, NGRP, pick_step, 0)

    def acc_step(g, acc):
        iv = idx_v[pl.ds(g * L, L)]
        lz = plsc.load_gather(logz_v, [iv])
        pk = picked_v[pl.ds(g * L, L)]
        return acc + (lz - pk)

    acc = lax.fori_loop(0, PER_W // L, acc_step, jnp.zeros((L,), jnp.float32))
    acc_v[...] = acc
    pltpu.sync_copy(acc_v, part_hbm.at[wid])


def _sc_loss(table_flat, idx_flat, tgt_flat, logz):
    mesh = plsc.VectorSubcoreMesh(core_axis_name="c", subcore_axis_name="s")
    f = pl.kernel(
        _loss_body,
        out_type=jax.ShapeDtypeStruct((NW, L), jnp.float32),
        mesh=mesh,
        compiler_params=pltpu.CompilerParams(
            needs_layout_passes=False, use_tc_tiling_on_sc=False),
        scratch_types=[
            pltpu.VMEM((PER_W,), jnp.int32),
            pltpu.VMEM((PER_W,), jnp.int32),
            pltpu.VMEM((PER_W,), jnp.int32),
            pltpu.VMEM((PER_W,), jnp.float32),
            pltpu.VMEM((VOCAB,), jnp.float32),
            pltpu.VMEM((L,), jnp.float32),
            pltpu.SemaphoreType.DMA,
        ],
    )
    return f(table_flat, idx_flat, tgt_flat, logz)


TRIM_R = 512


def _trim_body(x_ref, o_ref):
    o_ref[...] = x_ref[:, :VOCAB]


def _tc_trim(outp):
    return pl.pallas_call(
        _trim_body,
        grid=(BT // TRIM_R,),
        in_specs=[pl.BlockSpec((TRIM_R, VPAD), lambda i: (i, 0))],
        out_specs=pl.BlockSpec((TRIM_R, VOCAB), lambda i: (i, 0)),
        out_shape=jax.ShapeDtypeStruct((BT, VOCAB), jnp.float32),
    )(outp)


def kernel(idx, targets, table):
    idx_flat = idx.reshape(-1).astype(jnp.int32)
    tgt_flat = targets.reshape(-1).astype(jnp.int32)
    tablep = jnp.pad(table, ((0, 0), (0, VPAD - VOCAB)))
    logz = _row_logsumexp(table)
    outp = _sc_gather(tablep, idx_flat)
    logits2 = _tc_trim(outp)
    partials = _sc_loss(table.reshape(-1), idx_flat, tgt_flat, logz)
    loss = jnp.sum(partials) / BT
    return (logits2, loss)


# speedup vs baseline: 2.0019x; 1.2549x over previous
import jax
import jax.numpy as jnp
from jax import lax
from jax.experimental import pallas as pl
from jax.experimental.pallas import tpu as pltpu
from jax.experimental.pallas import tpu_sc as plsc

VOCAB = 1000
VPAD = 1024
BT = 4096 * 50
NC, NS, L = 2, 16, 16
NW = NC * NS
PER_W = BT // NW
CHUNK = 32
NCHUNK = PER_W // CHUNK
NBUF = 2


def _logz_body(t_ref, o_ref):
    x = t_ref[...]
    m = jnp.max(x, axis=1)
    s = jnp.sum(jnp.exp(x - m[:, None]), axis=1)
    o_ref[...] = m + jnp.log(s)


def _row_logsumexp(table):
    return pl.pallas_call(
        _logz_body,
        out_shape=jax.ShapeDtypeStruct((VOCAB,), jnp.float32),
    )(table)


def _gather_body(tablep_hbm, idx_hbm, outp_hbm,
                 idx_v, rows0_v, rows1_v, sem0, sem1):
    wid = lax.axis_index("s") * NC + lax.axis_index("c")
    base = wid * PER_W
    rows_bufs = (rows0_v, rows1_v)
    sems = (sem0, sem1)

    pltpu.sync_copy(idx_hbm.at[pl.ds(base, PER_W)], idx_v)

    def gather(c, b):
        return pltpu.async_copy(
            tablep_hbm.at[idx_v.at[pl.ds(c * CHUNK, CHUNK)]],
            rows_bufs[b], sems[b])

    for b in range(NBUF):
        gather(b, b)

    def outer_step(g, carry):
        for b in range(NBUF):
            c = g * NBUF + b
            rows_v = rows_bufs[b]
            pltpu.make_async_copy(
                tablep_hbm.at[idx_v.at[pl.ds(c * CHUNK, CHUNK)]],
                rows_v, sems[b]).wait()
            pltpu.sync_copy(rows_v, outp_hbm.at[pl.ds(base + c * CHUNK, CHUNK)])

            @pl.when(c + NBUF < NCHUNK)
            def _():
                gather(c + NBUF, b)
        return carry

    lax.fori_loop(0, NCHUNK // NBUF, outer_step, 0)


def _sc_gather(tablep, idx_flat):
    mesh = plsc.VectorSubcoreMesh(core_axis_name="c", subcore_axis_name="s")
    f = pl.kernel(
        _gather_body,
        out_type=jax.ShapeDtypeStruct((BT, VPAD), jnp.float32),
        mesh=mesh,
        compiler_params=pltpu.CompilerParams(needs_layout_passes=False),
        scratch_types=[
            pltpu.VMEM((PER_W,), jnp.int32),
            pltpu.VMEM((CHUNK, VPAD), jnp.float32),
            pltpu.VMEM((CHUNK, VPAD), jnp.float32),
            pltpu.SemaphoreType.DMA,
            pltpu.SemaphoreType.DMA,
        ],
    )
    return f(tablep, idx_flat)


GRP = 128
NGRP = PER_W // GRP


def _loss_body(tflat_hbm, idx_hbm, tgt_hbm, logz_hbm, part_hbm,
               idx_v, tgt_v, fi_v, picked_v, logz_v, acc_v, sem):
    wid = lax.axis_index("s") * NC + lax.axis_index("c")
    base = wid * PER_W

    pltpu.sync_copy(idx_hbm.at[pl.ds(base, PER_W)], idx_v)
    pltpu.sync_copy(tgt_hbm.at[pl.ds(base, PER_W)], tgt_v)
    pltpu.sync_copy(logz_hbm, logz_v)

    def fi_step(g, carry):
        iv = idx_v[pl.ds(g * L, L)]
        tv = tgt_v[pl.ds(g * L, L)]
        fi_v[pl.ds(g * L, L)] = iv * VOCAB + tv
        return carry

    lax.fori_loop(0, PER_W // L, fi_step, 0)

    def pick_step(k, carry):
        pltpu.async_copy(
            tflat_hbm.at[fi_v.at[pl.ds(k * GRP, GRP)]],
            picked_v.at[pl.ds(k * GRP, GRP)], sem).wait()
        return carry

    lax.fori_loop(0, NGRP, pick_step, 0)

    def acc_step(g, acc):
        iv = idx_v[pl.ds(g * L, L)]
        lz = plsc.load_gather(logz_v, [iv])
        pk = picked_v[pl.ds(g * L, L)]
        return acc + (lz - pk)

    acc = lax.fori_loop(0, PER_W // L, acc_step, jnp.zeros((L,), jnp.float32))
    acc_v[...] = acc
    pltpu.sync_copy(acc_v, part_hbm.at[wid])


def _sc_loss(table_flat, idx_flat, tgt_flat, logz):
    mesh = plsc.VectorSubcoreMesh(core_axis_name="c", subcore_axis_name="s")
    f = pl.kernel(
        _loss_body,
        out_type=jax.ShapeDtypeStruct((NW, L), jnp.float32),
        mesh=mesh,
        compiler_params=pltpu.CompilerParams(
            needs_layout_passes=False, use_tc_tiling_on_sc=False),
        scratch_types=[
            pltpu.VMEM((PER_W,), jnp.int32),
            pltpu.VMEM((PER_W,), jnp.int32),
            pltpu.VMEM((PER_W,), jnp.int32),
            pltpu.VMEM((PER_W,), jnp.float32),
            pltpu.VMEM((VOCAB,), jnp.float32),
            pltpu.VMEM((L,), jnp.float32),
            pltpu.SemaphoreType.DMA,
        ],
    )
    return f(table_flat, idx_flat, tgt_flat, logz)


TRIM_R = 256


def _trim_body(x_ref, o_ref):
    o_ref[...] = x_ref[:, :VOCAB].T


def _tc_trim(outp):
    return pl.pallas_call(
        _trim_body,
        grid=(BT // TRIM_R,),
        in_specs=[pl.BlockSpec((TRIM_R, VPAD), lambda i: (i, 0))],
        out_specs=pl.BlockSpec((VOCAB, TRIM_R), lambda i: (0, i)),
        out_shape=jax.ShapeDtypeStruct((VOCAB, BT), jnp.float32),
    )(outp)


def kernel(idx, targets, table):
    idx_flat = idx.reshape(-1).astype(jnp.int32)
    tgt_flat = targets.reshape(-1).astype(jnp.int32)
    tablep = jnp.pad(table, ((0, 0), (0, VPAD - VOCAB)))
    logz = _row_logsumexp(table)
    outp = _sc_gather(tablep, idx_flat)
    logits2 = _tc_trim(outp).T
    partials = _sc_loss(table.reshape(-1), idx_flat, tgt_flat, logz)
    loss = jnp.sum(partials) / BT
    return (logits2, loss)


# speedup vs baseline: 2.3704x; 1.1840x over previous
import jax
import jax.numpy as jnp
from jax import lax
from jax.experimental import pallas as pl
from jax.experimental.pallas import tpu as pltpu
from jax.experimental.pallas import tpu_sc as plsc

VOCAB = 1000
VPAD = 1024
BT = 4096 * 50
NC, NS, L = 2, 16, 16
NW = NC * NS
PER_W = BT // NW
CHUNK = 32
NCHUNK = PER_W // CHUNK
NBUF = 2


def _logz_body(t_ref, o_ref):
    x = t_ref[...]
    m = jnp.max(x, axis=1)
    s = jnp.sum(jnp.exp(x - m[:, None]), axis=1)
    o_ref[...] = m + jnp.log(s)


def _row_logsumexp(table):
    return pl.pallas_call(
        _logz_body,
        out_shape=jax.ShapeDtypeStruct((VOCAB,), jnp.float32),
    )(table)


def _gather_body(tablep_hbm, idx_hbm, outp_hbm,
                 idx_v, rows0_v, rows1_v, sem0, sem1):
    wid = lax.axis_index("s") * NC + lax.axis_index("c")
    base = wid * PER_W
    rows_bufs = (rows0_v, rows1_v)
    sems = (sem0, sem1)

    pltpu.sync_copy(idx_hbm.at[pl.ds(base, PER_W)], idx_v)

    def gather(c, b):
        return pltpu.async_copy(
            tablep_hbm.at[idx_v.at[pl.ds(c * CHUNK, CHUNK)]],
            rows_bufs[b], sems[b])

    for b in range(NBUF):
        gather(b, b)

    def outer_step(g, carry):
        for b in range(NBUF):
            c = g * NBUF + b
            rows_v = rows_bufs[b]
            pltpu.make_async_copy(
                tablep_hbm.at[idx_v.at[pl.ds(c * CHUNK, CHUNK)]],
                rows_v, sems[b]).wait()
            pltpu.sync_copy(rows_v, outp_hbm.at[pl.ds(base + c * CHUNK, CHUNK)])

            @pl.when(c + NBUF < NCHUNK)
            def _():
                gather(c + NBUF, b)
        return carry

    lax.fori_loop(0, NCHUNK // NBUF, outer_step, 0)


def _sc_gather(tablep, idx_flat):
    mesh = plsc.VectorSubcoreMesh(core_axis_name="c", subcore_axis_name="s")
    f = pl.kernel(
        _gather_body,
        out_type=jax.ShapeDtypeStruct((BT, VPAD), jnp.float32),
        mesh=mesh,
        compiler_params=pltpu.CompilerParams(needs_layout_passes=False),
        scratch_types=[
            pltpu.VMEM((PER_W,), jnp.int32),
            pltpu.VMEM((CHUNK, VPAD), jnp.float32),
            pltpu.VMEM((CHUNK, VPAD), jnp.float32),
            pltpu.SemaphoreType.DMA,
            pltpu.SemaphoreType.DMA,
        ],
    )
    return f(tablep, idx_flat)


GRP = 128
NGRP = PER_W // GRP


def _loss_body(tflat_hbm, idx_hbm, tgt_hbm, logz_hbm, part_hbm,
               idx_v, tgt_v, fi_v, picked_v, logz_v, acc_v, sem):
    wid = lax.axis_index("s") * NC + lax.axis_index("c")
    base = wid * PER_W

    pltpu.sync_copy(idx_hbm.at[pl.ds(base, PER_W)], idx_v)
    pltpu.sync_copy(tgt_hbm.at[pl.ds(base, PER_W)], tgt_v)
    pltpu.sync_copy(logz_hbm, logz_v)

    def fi_step(g, carry):
        iv = idx_v[pl.ds(g * L, L)]
        tv = tgt_v[pl.ds(g * L, L)]
        fi_v[pl.ds(g * L, L)] = iv * VOCAB + tv
        return carry

    lax.fori_loop(0, PER_W // L, fi_step, 0)

    def pick_step(k, carry):
        pltpu.async_copy(
            tflat_hbm.at[fi_v.at[pl.ds(k * GRP, GRP)]],
            picked_v.at[pl.ds(k * GRP, GRP)], sem).wait()
        return carry

    lax.fori_loop(0, NGRP, pick_step, 0)

    def acc_step(g, acc):
        iv = idx_v[pl.ds(g * L, L)]
        lz = plsc.load_gather(logz_v, [iv])
        pk = picked_v[pl.ds(g * L, L)]
        return acc + (lz - pk)

    acc = lax.fori_loop(0, PER_W // L, acc_step, jnp.zeros((L,), jnp.float32))
    acc_v[...] = acc
    pltpu.sync_copy(acc_v, part_hbm.at[wid])


def _sc_loss(table_flat, idx_flat, tgt_flat, logz):
    mesh = plsc.VectorSubcoreMesh(core_axis_name="c", subcore_axis_name="s")
    f = pl.kernel(
        _loss_body,
        out_type=jax.ShapeDtypeStruct((NW, L), jnp.float32),
        mesh=mesh,
        compiler_params=pltpu.CompilerParams(
            needs_layout_passes=False, use_tc_tiling_on_sc=False),
        scratch_types=[
            pltpu.VMEM((PER_W,), jnp.int32),
            pltpu.VMEM((PER_W,), jnp.int32),
            pltpu.VMEM((PER_W,), jnp.int32),
            pltpu.VMEM((PER_W,), jnp.float32),
            pltpu.VMEM((VOCAB,), jnp.float32),
            pltpu.VMEM((L,), jnp.float32),
            pltpu.SemaphoreType.DMA,
        ],
    )
    return f(table_flat, idx_flat, tgt_flat, logz)


TRIM_R = 512


def _trim_body(x_ref, o_ref):
    o_ref[...] = x_ref[...].T[:VOCAB, :]


def _tc_trim(outp):
    return pl.pallas_call(
        _trim_body,
        grid=(BT // TRIM_R,),
        in_specs=[pl.BlockSpec((TRIM_R, VPAD), lambda i: (i, 0))],
        out_specs=pl.BlockSpec((VOCAB, TRIM_R), lambda i: (0, i)),
        out_shape=jax.ShapeDtypeStruct((VOCAB, BT), jnp.float32),
    )(outp)


def kernel(idx, targets, table):
    idx_flat = idx.reshape(-1).astype(jnp.int32)
    tgt_flat = targets.reshape(-1).astype(jnp.int32)
    tablep = jnp.pad(table, ((0, 0), (0, VPAD - VOCAB)))
    logz = _row_logsumexp(table)
    outp = _sc_gather(tablep, idx_flat)
    logits2 = _tc_trim(outp).T
    partials = _sc_loss(table.reshape(-1), idx_flat, tgt_flat, logz)
    loss = jnp.sum(partials) / BT
    return (logits2, loss)
